# baseline (speedup 1.0000x reference)
import jax
import jax.numpy as jnp
from jax.experimental import pallas as pl
from jax.experimental.pallas import tpu as pltpu

_CHUNK = 1000
_NBUF = 10
_LOOKAHEAD = 10
_D = 128


def _mlp_body(x_hbm, w1_ref, b1_ref, w2_ref, b2_ref, o_hbm,
              ibuf, obuf, in_sem, out_sem):
    n_chunks = x_hbm.shape[0] // _CHUNK

    def in_copy(k):
        s = k % _NBUF
        return pltpu.make_async_copy(
            x_hbm.at[pl.ds(k * _CHUNK, _CHUNK), :], ibuf.at[s], in_sem.at[s])

    def out_copy(k):
        s = k % _NBUF
        return pltpu.make_async_copy(
            obuf.at[s], o_hbm.at[pl.ds(k * _CHUNK, _CHUNK), :], out_sem.at[s])

    for k in range(min(_LOOKAHEAD, n_chunks)):
        in_copy(k).start()
    for k in range(n_chunks):
        s = k % _NBUF
        if k + _LOOKAHEAD < n_chunks:
            in_copy(k + _LOOKAHEAD).start()
        in_copy(k).wait()
        if k >= _NBUF:
            out_copy(k - _NBUF).wait()
        h = jnp.dot(ibuf[s].astype(jnp.bfloat16),
                    w1_ref[:].astype(jnp.bfloat16),
                    preferred_element_type=jnp.float32)
        h = h + b1_ref[:]
        h = jnp.where(h > 0, h, jnp.exp(h) - 1.0)
        o = jnp.dot(h.astype(jnp.bfloat16),
                    w2_ref[:].astype(jnp.bfloat16),
                    preferred_element_type=jnp.float32)
        obuf[s] = o + b2_ref[:]
        out_copy(k).start()
    for k in range(max(n_chunks - _NBUF, 0), n_chunks):
        out_copy(k).wait()


def kernel(x, edge_index, W1, b1, W2, b2):
    del edge_index
    N, D = x.shape
    b1_2d = b1.reshape(1, D)
    b2_2d = b2.reshape(1, D)
    return pl.pallas_call(
        _mlp_body,
        in_specs=[
            pl.BlockSpec(memory_space=pltpu.MemorySpace.HBM),
            pl.BlockSpec(memory_space=pltpu.MemorySpace.VMEM),
            pl.BlockSpec(memory_space=pltpu.MemorySpace.VMEM),
            pl.BlockSpec(memory_space=pltpu.MemorySpace.VMEM),
            pl.BlockSpec(memory_space=pltpu.MemorySpace.VMEM),
        ],
        out_specs=pl.BlockSpec(memory_space=pltpu.MemorySpace.HBM),
        out_shape=jax.ShapeDtypeStruct((N, D), x.dtype),
        scratch_shapes=[
            pltpu.VMEM((_NBUF, _CHUNK, _D), jnp.float32),
            pltpu.VMEM((_NBUF, _CHUNK, _D), jnp.float32),
            pltpu.SemaphoreType.DMA((_NBUF,)),
            pltpu.SemaphoreType.DMA((_NBUF,)),
        ],
    )(x, W1, b1_2d, W2, b2_2d)
